# baseline (speedup 1.0000x reference)
import jax
import jax.numpy as jnp
from jax.experimental import pallas as pl

_BS = 256


def _add_body(tok_ref, pos_ref, out_ref):
    out_ref[...] = tok_ref[...] + pos_ref[...][None, :, :]


def kernel(token_embeddings, pos_table):
    batch, seq, dim = token_embeddings.shape
    return pl.pallas_call(
        _add_body,
        grid=(seq // _BS,),
        in_specs=[
            pl.BlockSpec((batch, _BS, dim), lambda s: (0, s, 0)),
            pl.BlockSpec((_BS, dim), lambda s: (s, 0)),
        ],
        out_specs=pl.BlockSpec((batch, _BS, dim), lambda s: (0, s, 0)),
        out_shape=jax.ShapeDtypeStruct((batch, seq, dim), token_embeddings.dtype),
    )(token_embeddings, pos_table)
